# Initial kernel scaffold; baseline (speedup 1.0000x reference)
import jax
import jax.numpy as jnp
from jax.experimental import pallas as pl


def kernel(x, edge_index, edge_attr, batch, We, be, W1, Wr1, b1, W2, Wr2, b2, W3, Wr3, b3, Wfc, bfc):
    raise NotImplementedError("write your pallas kernel here")


# speedup vs baseline: 3.3103x; 3.3103x over previous
import functools

import jax
import jax.numpy as jnp
from jax import lax
from jax.experimental import pallas as pl
from jax.experimental.pallas import tpu as pltpu
from jax.experimental.pallas import tpu_sc as plsc

NC = 2
NS = 16
NW = NC * NS
LANES = 16
G = 64


def _gate_body(ea_ref, we_ref, be_ref, out_ref):
    out_ref[...] = jnp.maximum(
        jnp.dot(ea_ref[...], we_ref[...], preferred_element_type=jnp.float32)
        + be_ref[...], 0.0)


def _tc_gate(edge_attr, We, be2):
    E, DE = edge_attr.shape
    D = We.shape[1]
    BE = 1600
    return pl.pallas_call(
        _gate_body,
        grid=(E // BE,),
        in_specs=[
            pl.BlockSpec((BE, DE), lambda i: (i, 0)),
            pl.BlockSpec((DE, D), lambda i: (0, 0)),
            pl.BlockSpec((1, D), lambda i: (0, 0)),
        ],
        out_specs=pl.BlockSpec((BE, D), lambda i: (i, 0)),
        out_shape=jax.ShapeDtypeStruct((E, D), jnp.float32),
    )(edge_attr, We, be2)


def _proj2_body(x_ref, w_ref, wr_ref, b_ref, y_ref, xr_ref):
    xb = x_ref[...]
    y_ref[...] = jnp.dot(xb, w_ref[...], preferred_element_type=jnp.float32)
    xr_ref[...] = (
        jnp.dot(xb, wr_ref[...], preferred_element_type=jnp.float32)
        + b_ref[...])


def _tc_proj2(x, W, Wr, b2):
    N, D = x.shape
    BN = 1000
    return pl.pallas_call(
        _proj2_body,
        grid=(N // BN,),
        in_specs=[
            pl.BlockSpec((BN, D), lambda i: (i, 0)),
            pl.BlockSpec((D, D), lambda i: (0, 0)),
            pl.BlockSpec((D, D), lambda i: (0, 0)),
            pl.BlockSpec((1, D), lambda i: (0, 0)),
        ],
        out_specs=[
            pl.BlockSpec((BN, D), lambda i: (i, 0)),
            pl.BlockSpec((BN, D), lambda i: (i, 0)),
        ],
        out_shape=[
            jax.ShapeDtypeStruct((N, D), jnp.float32),
            jax.ShapeDtypeStruct((N, D), jnp.float32),
        ],
    )(x, W, Wr, b2)


def _combine_body(agg_ref, xr_ref, w_ref, wr_ref, b_ref, y_ref, hr_ref):
    h = jnp.maximum(agg_ref[0] + agg_ref[1] + xr_ref[...], 0.0)
    y_ref[...] = jnp.dot(h, w_ref[...], preferred_element_type=jnp.float32)
    hr_ref[...] = (
        jnp.dot(h, wr_ref[...], preferred_element_type=jnp.float32)
        + b_ref[...])


def _tc_combine(aggp, xr, W, Wr, b2):
    N, D = xr.shape
    BN = 1000
    return pl.pallas_call(
        _combine_body,
        grid=(N // BN,),
        in_specs=[
            pl.BlockSpec((NC, BN, D), lambda i: (0, i, 0)),
            pl.BlockSpec((BN, D), lambda i: (i, 0)),
            pl.BlockSpec((D, D), lambda i: (0, 0)),
            pl.BlockSpec((D, D), lambda i: (0, 0)),
            pl.BlockSpec((1, D), lambda i: (0, 0)),
        ],
        out_specs=[
            pl.BlockSpec((BN, D), lambda i: (i, 0)),
            pl.BlockSpec((BN, D), lambda i: (i, 0)),
        ],
        out_shape=[
            jax.ShapeDtypeStruct((N, D), jnp.float32),
            jax.ShapeDtypeStruct((N, D), jnp.float32),
        ],
    )(aggp, xr, W, Wr, b2)


def _final_body(agg_ref, hr_ref, batch_ref, wfc_ref, bfc_ref, out_ref,
                sums_ref, cnts_ref):
    i = pl.program_id(0)
    nb = pl.num_programs(0)

    @pl.when(i == 0)
    def _():
        sums_ref[...] = jnp.zeros_like(sums_ref)
        cnts_ref[...] = jnp.zeros_like(cnts_ref)

    h = jnp.maximum(agg_ref[0] + agg_ref[1] + hr_ref[...], 0.0)
    z = jnp.dot(h, wfc_ref[...], preferred_element_type=jnp.float32)
    gids = lax.broadcasted_iota(jnp.int32, (1, G), 1).astype(jnp.float32)
    mask = (batch_ref[...] == gids).astype(jnp.float32)
    sums_ref[...] += jnp.sum(mask * z, axis=0, keepdims=True)
    cnts_ref[...] += jnp.sum(mask, axis=0, keepdims=True)

    @pl.when(i == nb - 1)
    def _():
        out_ref[...] = (sums_ref[...] / jnp.maximum(cnts_ref[...], 1.0)
                        + bfc_ref[...])


def _tc_final(aggp, hr, batch_f, Wfc, bfc2):
    N, D = hr.shape
    BN = 1000
    return pl.pallas_call(
        _final_body,
        grid=(N // BN,),
        in_specs=[
            pl.BlockSpec((NC, BN, D), lambda i: (0, i, 0)),
            pl.BlockSpec((BN, D), lambda i: (i, 0)),
            pl.BlockSpec((BN, 1), lambda i: (i, 0)),
            pl.BlockSpec((D, 1), lambda i: (0, 0)),
            pl.BlockSpec((1, 1), lambda i: (0, 0)),
        ],
        out_specs=pl.BlockSpec((1, G), lambda i: (0, 0)),
        out_shape=jax.ShapeDtypeStruct((1, G), jnp.float32),
        scratch_shapes=[
            pltpu.VMEM((1, G), jnp.float32),
            pltpu.VMEM((1, G), jnp.float32),
        ],
    )(aggp, hr, batch_f, Wfc, bfc2)


def _make_sc_segsum(N, E, D, gated):
    EPW = E // NW
    C = 80
    NCH = EPW // C
    NP = ((N + 8 * NS - 1) // (8 * NS)) * (8 * NS)
    RPT = NP // NS
    mesh = plsc.VectorSubcoreMesh(
        core_axis_name="c", subcore_axis_name="s",
        num_cores=NC, num_subcores=NS)

    scratch = [
        pltpu.VMEM_SHARED((NP, D), jnp.float32),
        pltpu.VMEM((C,), jnp.int32),
        pltpu.VMEM((C,), jnp.int32),
        pltpu.VMEM((C, D), jnp.float32),
    ]
    if gated:
        scratch.append(pltpu.VMEM((C, D), jnp.float32))
    scratch.append(pltpu.SemaphoreType.DMA)

    def body(*refs):
        if gated:
            (y_hbm, src_hbm, dst_hbm, gate_hbm, zeros_hbm, out_hbm,
             accum, src_v, dst_v, rows_v, gate_v, sem) = refs
        else:
            (y_hbm, src_hbm, dst_hbm, zeros_hbm, out_hbm,
             accum, src_v, dst_v, rows_v, sem) = refs
        c = lax.axis_index("c")
        s = lax.axis_index("s")
        wid = c * NS + s
        pltpu.sync_copy(zeros_hbm, accum.at[pl.ds(s * RPT, RPT), :])
        plsc.subcore_barrier()
        base = wid * EPW

        def chunk(i, carry):
            off = base + i * C
            pltpu.sync_copy(src_hbm.at[pl.ds(off, C)], src_v)
            pltpu.sync_copy(dst_hbm.at[pl.ds(off, C)], dst_v)
            pltpu.async_copy(y_hbm.at[src_v], rows_v, sem).wait()
            if gated:
                pltpu.sync_copy(gate_hbm.at[pl.ds(off, C), :], gate_v)

                def mul_row(r, cc):
                    for k in range(D // LANES):
                        sl = pl.ds(k * LANES, LANES)
                        rows_v[r, sl] = rows_v[r, sl] * gate_v[r, sl]
                    return cc

                lax.fori_loop(0, C, mul_row, 0)
            pltpu.sync_copy(rows_v, accum.at[dst_v], add=True)
            return carry

        lax.fori_loop(0, NCH, chunk, 0)
        plsc.subcore_barrier()
        sl = pl.ds(s * RPT, RPT)
        pltpu.sync_copy(accum.at[sl, :], out_hbm.at[c, sl, :])

    return pl.kernel(
        body,
        out_type=jax.ShapeDtypeStruct((NC, NP, D), jnp.float32),
        mesh=mesh,
        scratch_types=scratch,
    )


def kernel(x, edge_index, edge_attr, batch, We, be, W1, Wr1, b1,
           W2, Wr2, b2, W3, Wr3, b3, Wfc, bfc):
    N, D = x.shape
    E = edge_index.shape[1]
    src = edge_index[0]
    dst = edge_index[1]
    NP = ((N + 8 * NS - 1) // (8 * NS)) * (8 * NS)
    zeros = jnp.zeros((NP // NS, D), jnp.float32)

    gate = _tc_gate(edge_attr, We, be.reshape(1, D))
    y1, xr1 = _tc_proj2(x, W1, Wr1, b1.reshape(1, D))
    agg1 = _make_sc_segsum(N, E, D, True)(y1, src, dst, gate, zeros)
    y2, hr2 = _tc_combine(agg1, xr1, W2, Wr2, b2.reshape(1, D))
    agg2 = _make_sc_segsum(N, E, D, False)(y2, src, dst, zeros)
    y3, hr3 = _tc_combine(agg2, hr2, W3, Wr3, b3.reshape(1, D))
    agg3 = _make_sc_segsum(N, E, D, False)(y3, src, dst, zeros)
    batch_f = batch.astype(jnp.float32).reshape(N, 1)
    out = _tc_final(agg3, hr3, batch_f, Wfc, bfc.reshape(1, 1))
    return out.reshape(G)
